# baseline (speedup 1.0000x reference)
import functools

import jax
import jax.numpy as jnp
from jax import lax
from jax.experimental import pallas as pl
from jax.experimental.pallas import tpu as pltpu
from jax.experimental.pallas import tpu_sc as plsc

BATCH, SEQ = 4096, 20
NTOK = BATCH * SEQ
V = 1000
VP = 1024
D = 4
L = 16

NC, NS = 2, 16
NW = NC * NS
BPW = BATCH // NW
TPW = BPW * SEQ
NFULL = V // L
TAIL = V - L


def _table_body(embed_ref, w_ref, b_ref, table_ref):
    table_ref[...] = lax.dot_general(
        embed_ref[...], w_ref[...],
        dimension_numbers=(((1,), (1,)), ((), ())),
        preferred_element_type=jnp.float32) + b_ref[...]


@functools.partial(
    pl.kernel,
    out_type=jax.ShapeDtypeStruct((BATCH, SEQ, V), jnp.float32),
    mesh=plsc.VectorSubcoreMesh(core_axis_name="c", subcore_axis_name="s"),
    scratch_types=[
        pltpu.VMEM((TPW,), jnp.int32),
        pltpu.VMEM((2 * SEQ, VP), jnp.float32),
        pltpu.VMEM((SEQ, V), jnp.float32),
        pltpu.SemaphoreType.DMA,
    ],
)
def _sc_gather(idx_hbm, table_hbm, out_hbm, idx_v, rows_v, batch_v, sem):
    wid = lax.axis_index("s") * NC + lax.axis_index("c")
    pltpu.sync_copy(idx_hbm.at[wid], idx_v)

    def repack_row(r, p):
        src = p * SEQ + r
        for k in range(NFULL):
            batch_v[r, pl.ds(k * L, L)] = rows_v[src, pl.ds(k * L, L)]
        batch_v[r, pl.ds(TAIL, L)] = rows_v[src, pl.ds(TAIL, L)]
        return p

    def step(i, carry):
        pltpu.async_copy(
            table_hbm.at[idx_v.at[pl.ds(i * 2 * SEQ, 2 * SEQ)]],
            rows_v, sem).wait()
        for p in range(2):
            lax.fori_loop(0, SEQ, repack_row, p)
            pltpu.sync_copy(batch_v, out_hbm.at[wid * BPW + 2 * i + p])
        return carry

    lax.fori_loop(0, BPW // 2, step, 0)


def kernel(x, embed, W, b):
    w_pad = jnp.pad(W, ((0, VP - V), (0, 0)))
    b_pad = jnp.pad(b, (0, VP - V)).reshape(1, VP)
    table = pl.pallas_call(
        _table_body,
        out_shape=jax.ShapeDtypeStruct((V, VP), jnp.float32),
    )(embed, w_pad, b_pad)
    idx = x.astype(jnp.int32).reshape(NW, TPW)
    return _sc_gather(idx, table)


# speedup vs baseline: 1.8824x; 1.8824x over previous
import functools

import jax
import jax.numpy as jnp
from jax import lax
from jax.experimental import pallas as pl
from jax.experimental.pallas import tpu as pltpu
from jax.experimental.pallas import tpu_sc as plsc

BATCH, SEQ = 4096, 20
NTOK = BATCH * SEQ
V = 1000
D = 4
DP = 16

NC, NS = 2, 16
NW = NC * NS
TPW = NTOK // NW
IDXC = 128
NCH = TPW // IDXC

BB = 64
NBLK = BATCH // BB


@functools.partial(
    pl.kernel,
    out_type=jax.ShapeDtypeStruct((NTOK, DP), jnp.float32),
    mesh=plsc.VectorSubcoreMesh(core_axis_name="c", subcore_axis_name="s"),
    compiler_params=pltpu.CompilerParams(use_tc_tiling_on_sc=False),
    scratch_types=[
        pltpu.VMEM((TPW,), jnp.int32),
        pltpu.VMEM((TPW, DP), jnp.float32),
        pltpu.SemaphoreType.DMA,
    ],
)
def _sc_gather(idx_hbm, embed_hbm, emb_hbm, idx_v, rows_v, sem):
    wid = lax.axis_index("s") * NC + lax.axis_index("c")
    base = wid * TPW
    pltpu.sync_copy(idx_hbm.at[pl.ds(base, TPW)], idx_v)
    descs = [
        pltpu.async_copy(
            embed_hbm.at[idx_v.at[pl.ds(c * IDXC, IDXC)]],
            rows_v.at[pl.ds(c * IDXC, IDXC)], sem)
        for c in range(NCH)
    ]
    for d in descs:
        d.wait()
    pltpu.sync_copy(rows_v, emb_hbm.at[pl.ds(base, TPW)])


def _mm_body(emb_ref, w_ref, b_ref, out_ref):
    e = emb_ref[...].reshape(BB * SEQ, DP)
    out = lax.dot_general(
        e, w_ref[...],
        dimension_numbers=(((1,), (0,)), ((), ())),
        preferred_element_type=jnp.float32) + b_ref[...]
    out_ref[...] = out.reshape(BB, SEQ, V)


def kernel(x, embed, W, b):
    embed16 = jnp.pad(embed, ((0, 0), (0, DP - D)))
    w16 = jnp.pad(W.T, ((0, DP - D), (0, 0)))
    emb = _sc_gather(x.astype(jnp.int32).reshape(NTOK), embed16)
    out = pl.pallas_call(
        _mm_body,
        grid=(NBLK,),
        in_specs=[
            pl.BlockSpec((BB, SEQ, DP), lambda i: (i, 0, 0)),
            pl.BlockSpec((DP, V), lambda i: (0, 0)),
            pl.BlockSpec((1, V), lambda i: (0, 0)),
        ],
        out_specs=pl.BlockSpec((BB, SEQ, V), lambda i: (i, 0, 0)),
        out_shape=jax.ShapeDtypeStruct((BATCH, SEQ, V), jnp.float32),
    )(emb.reshape(BATCH, SEQ, DP), w16, b.reshape(1, V))
    return out


# speedup vs baseline: 1.8902x; 1.0042x over previous
import functools

import jax
import jax.numpy as jnp
from jax import lax
from jax.experimental import pallas as pl
from jax.experimental.pallas import tpu as pltpu
from jax.experimental.pallas import tpu_sc as plsc

BATCH, SEQ = 4096, 20
NTOK = BATCH * SEQ
V = 1000
D = 4
DP = 16

NC, NS = 2, 16
NW = NC * NS
TPW = NTOK // NW
IDXC = 128
NCH = TPW // IDXC

BB = 128
NBLK = BATCH // BB


@functools.partial(
    pl.kernel,
    out_type=jax.ShapeDtypeStruct((NTOK, DP), jnp.float32),
    mesh=plsc.VectorSubcoreMesh(core_axis_name="c", subcore_axis_name="s"),
    compiler_params=pltpu.CompilerParams(use_tc_tiling_on_sc=False),
    scratch_types=[
        pltpu.VMEM((TPW,), jnp.int32),
        pltpu.VMEM((TPW, DP), jnp.float32),
        pltpu.SemaphoreType.DMA,
    ],
)
def _sc_gather(idx_hbm, embed_hbm, emb_hbm, idx_v, rows_v, sem):
    wid = lax.axis_index("s") * NC + lax.axis_index("c")
    base = wid * TPW
    pltpu.sync_copy(idx_hbm.at[pl.ds(base, TPW)], idx_v)
    descs = [
        pltpu.async_copy(
            embed_hbm.at[idx_v.at[pl.ds(c * IDXC, IDXC)]],
            rows_v.at[pl.ds(c * IDXC, IDXC)], sem)
        for c in range(NCH)
    ]
    for d in descs:
        d.wait()
    pltpu.sync_copy(rows_v, emb_hbm.at[pl.ds(base, TPW)])


def _mm_body(emb_ref, w_ref, b_ref, out_ref):
    e = emb_ref[...].reshape(BB * SEQ, DP)
    out = lax.dot_general(
        e, w_ref[...],
        dimension_numbers=(((1,), (0,)), ((), ())),
        preferred_element_type=jnp.float32) + b_ref[...]
    out_ref[...] = out.reshape(BB, SEQ, V)


def kernel(x, embed, W, b):
    embed16 = jnp.pad(embed, ((0, 0), (0, DP - D)))
    w16 = jnp.pad(W.T, ((0, DP - D), (0, 0)))
    emb = _sc_gather(x.astype(jnp.int32).reshape(NTOK), embed16)
    out = pl.pallas_call(
        _mm_body,
        grid=(NBLK,),
        in_specs=[
            pl.BlockSpec((BB, SEQ, DP), lambda i: (i, 0, 0)),
            pl.BlockSpec((DP, V), lambda i: (0, 0)),
            pl.BlockSpec((1, V), lambda i: (0, 0)),
        ],
        out_specs=pl.BlockSpec((BB, SEQ, V), lambda i: (i, 0, 0)),
        out_shape=jax.ShapeDtypeStruct((BATCH, SEQ, V), jnp.float32),
    )(emb.reshape(BATCH, SEQ, DP), w16, b.reshape(1, V))
    return out


# speedup vs baseline: 5.8302x; 3.0844x over previous
import functools

import jax
import jax.numpy as jnp
from jax import lax
from jax.experimental import pallas as pl
from jax.experimental.pallas import tpu as pltpu
from jax.experimental.pallas import tpu_sc as plsc

BATCH, SEQ = 4096, 20
NTOK = BATCH * SEQ
V = 1000
D = 4
DP = 16
PK = 128
PR = NTOK * DP // PK

NC, NS = 2, 16
NW = NC * NS
TPW = NTOK // NW
IDXC = 128
NCH = TPW // IDXC


@functools.partial(
    pl.kernel,
    out_type=jax.ShapeDtypeStruct((NTOK, DP), jnp.float32),
    mesh=plsc.VectorSubcoreMesh(core_axis_name="c", subcore_axis_name="s"),
    compiler_params=pltpu.CompilerParams(use_tc_tiling_on_sc=False),
    scratch_types=[
        pltpu.VMEM((TPW,), jnp.int32),
        pltpu.VMEM((TPW, DP), jnp.float32),
        pltpu.SemaphoreType.DMA,
    ],
)
def _sc_gather(idx_hbm, embed_hbm, emb_hbm, idx_v, rows_v, sem):
    wid = lax.axis_index("s") * NC + lax.axis_index("c")
    base = wid * TPW
    pltpu.sync_copy(idx_hbm.at[pl.ds(base, TPW)], idx_v)
    descs = [
        pltpu.async_copy(
            embed_hbm.at[idx_v.at[pl.ds(c * IDXC, IDXC)]],
            rows_v.at[pl.ds(c * IDXC, IDXC)], sem)
        for c in range(NCH)
    ]
    for d in descs:
        d.wait()
    pltpu.sync_copy(rows_v, emb_hbm.at[pl.ds(base, TPW)])


BB = 2048
NI = BATCH // BB


def _mm_body(emb_ref, w_ref, b_ref, out_ref):
    out_ref[...] = lax.dot_general(
        w_ref[...], emb_ref[...],
        dimension_numbers=(((1,), (1,)), ((), ())),
        preferred_element_type=jnp.float32) + b_ref[...]


def kernel(x, embed, W, b):
    embed16 = jnp.pad(embed, ((0, 0), (0, DP - D)))
    w16 = jnp.pad(W, ((0, 0), (0, DP - D)))
    idx = x.astype(jnp.int32).T.reshape(NTOK)
    emb = _sc_gather(idx, embed16)
    out_t = pl.pallas_call(
        _mm_body,
        grid=(SEQ, NI),
        in_specs=[
            pl.BlockSpec((BB, DP), lambda s, i: (s * NI + i, 0)),
            pl.BlockSpec((V, DP), lambda s, i: (0, 0)),
            pl.BlockSpec((V, 1), lambda s, i: (0, 0)),
        ],
        out_specs=pl.BlockSpec((None, V, BB), lambda s, i: (s, 0, i)),
        out_shape=jax.ShapeDtypeStruct((SEQ, V, BATCH), jnp.float32),
    )(emb, w16, b.reshape(V, 1))
    return out_t.transpose(2, 0, 1)


# speedup vs baseline: 5.9087x; 1.0135x over previous
import functools

import jax
import jax.numpy as jnp
from jax import lax
from jax.experimental import pallas as pl
from jax.experimental.pallas import tpu as pltpu
from jax.experimental.pallas import tpu_sc as plsc

BATCH, SEQ = 4096, 20
NTOK = BATCH * SEQ
V = 1000
D = 4
DP = 16
PK = 128
PR = NTOK * DP // PK

NC, NS = 2, 16
NW = NC * NS
TPW = NTOK // NW
IDXC = 128
NCH = TPW // IDXC


@functools.partial(
    pl.kernel,
    out_type=jax.ShapeDtypeStruct((NTOK, DP), jnp.float32),
    mesh=plsc.VectorSubcoreMesh(core_axis_name="c", subcore_axis_name="s"),
    compiler_params=pltpu.CompilerParams(use_tc_tiling_on_sc=False),
    scratch_types=[
        pltpu.VMEM((TPW,), jnp.int32),
        pltpu.VMEM((TPW, DP), jnp.float32),
        pltpu.SemaphoreType.DMA,
    ],
)
def _sc_gather(idx_hbm, embed_hbm, emb_hbm, idx_v, rows_v, sem):
    wid = lax.axis_index("s") * NC + lax.axis_index("c")
    base = wid * TPW
    pltpu.sync_copy(idx_hbm.at[pl.ds(base, TPW)], idx_v)
    descs = [
        pltpu.async_copy(
            embed_hbm.at[idx_v.at[pl.ds(c * IDXC, IDXC)]],
            rows_v.at[pl.ds(c * IDXC, IDXC)], sem)
        for c in range(NCH)
    ]
    for d in descs:
        d.wait()
    pltpu.sync_copy(rows_v, emb_hbm.at[pl.ds(base, TPW)])


BB = 4096
NI = BATCH // BB


def _mm_body(emb_ref, w_ref, b_ref, out_ref):
    out_ref[...] = lax.dot_general(
        w_ref[...], emb_ref[...],
        dimension_numbers=(((1,), (1,)), ((), ())),
        preferred_element_type=jnp.float32) + b_ref[...]


def kernel(x, embed, W, b):
    embed16 = jnp.pad(embed, ((0, 0), (0, DP - D)))
    w16 = jnp.pad(W, ((0, 0), (0, DP - D)))
    idx = x.astype(jnp.int32).T.reshape(NTOK)
    emb = _sc_gather(idx, embed16)
    out_t = pl.pallas_call(
        _mm_body,
        grid=(SEQ, NI),
        in_specs=[
            pl.BlockSpec((BB, DP), lambda s, i: (s * NI + i, 0)),
            pl.BlockSpec((V, DP), lambda s, i: (0, 0)),
            pl.BlockSpec((V, 1), lambda s, i: (0, 0)),
        ],
        out_specs=pl.BlockSpec((None, V, BB), lambda s, i: (s, 0, i)),
        out_shape=jax.ShapeDtypeStruct((SEQ, V, BATCH), jnp.float32),
    )(emb, w16, b.reshape(V, 1))
    return out_t.transpose(2, 0, 1)
